# baseline (speedup 1.0000x reference)
import jax
import jax.numpy as jnp
from jax.experimental import pallas as pl
from jax.experimental.pallas import tpu as pltpu

_H = 64


def _round_up(n, m):
    return ((n + m - 1) // m) * m


def _sigmoid_t(x):
    return 0.5 * jnp.tanh(x * 0.5) + 0.5


def _fused_kernel(x_ref, wg_ref, w1t_ref, w2_ref, b_ref, out_ref, *,
                  gate_rows, nh_rows, na_rows, tb, n_split):
    wg = wg_ref[...].astype(jnp.bfloat16)
    w2 = w2_ref[...].astype(jnp.bfloat16)

    b1row = jnp.transpose(b_ref[gate_rows:gate_rows + nh_rows, :], (1, 0))
    brows = jnp.where(
        jax.lax.broadcasted_iota(jnp.int32, (8, nh_rows), 0) == 0,
        b1row, 0.0).astype(jnp.bfloat16)
    w1taug = jnp.concatenate(
        [w1t_ref[...].astype(jnp.bfloat16), brows], axis=0)

    bg = b_ref[0:gate_rows, :]
    b2 = b_ref[gate_rows + nh_rows:gate_rows + nh_rows + na_rows, :]
    aug = (jax.lax.broadcasted_iota(jnp.int32, (8, tb // n_split), 0) == 0)
    aug = aug.astype(jnp.bfloat16)

    ts = tb // n_split
    for s in range(n_split):
        xb = x_ref[s * ts:(s + 1) * ts, :].astype(jnp.bfloat16)

        gates = jax.lax.dot_general(
            wg, xb, (((1,), (1,)), ((), ())),
            preferred_element_type=jnp.float32)
        gates = gates + bg

        i = _sigmoid_t(gates[0:_H, :])
        o = _sigmoid_t(gates[_H:2 * _H, :])
        g = jnp.tanh(gates[2 * _H:3 * _H, :])
        h = (o * jnp.tanh(i * g)).astype(jnp.bfloat16)

        haug = jnp.concatenate([h, aug], axis=0)

        h1 = jax.lax.dot_general(
            w1taug, haug, (((0,), (0,)), ((), ())),
            preferred_element_type=jnp.float32)
        h1 = jnp.maximum(h1.astype(jnp.bfloat16), jnp.bfloat16(0))

        out = jnp.dot(w2, h1, preferred_element_type=jnp.float32) + b2
        out_ref[:, s * ts:(s + 1) * ts] = out


def kernel(x, w_gate, w1, w2, b_pack):
    B, n_states = x.shape
    gate_rows, k_pad = w_gate.shape
    nh_rows = w1.shape[0]
    na_rows = w2.shape[0]
    btot = gate_rows + nh_rows + na_rows

    x32 = x.astype(jnp.float32)
    if k_pad != n_states:
        x32 = jnp.pad(x32, ((0, 0), (0, k_pad - n_states)))

    if B <= 1024:
        tb = _round_up(B, 8)
        n_split = 1
    else:
        tb = min(16384, _round_up(pl.cdiv(B, 2), 256))
        n_split = 8 if tb % 64 == 0 else 1
    grid_b = pl.cdiv(B, tb)

    def body(x_ref, wg_ref, w1t_ref, w2_ref, b_ref, out_ref):
        _fused_kernel(x_ref, wg_ref, w1t_ref, w2_ref, b_ref, out_ref,
                      gate_rows=gate_rows, nh_rows=nh_rows, na_rows=na_rows,
                      tb=tb, n_split=n_split)

    out = pl.pallas_call(
        body,
        out_shape=jax.ShapeDtypeStruct((na_rows, grid_b * tb), jnp.float32),
        grid=(grid_b,),
        in_specs=[
            pl.BlockSpec((tb, k_pad), lambda i: (i, 0)),
            pl.BlockSpec((gate_rows, k_pad), lambda i: (0, 0)),
            pl.BlockSpec((_H, nh_rows), lambda i: (0, 0)),
            pl.BlockSpec((na_rows, nh_rows), lambda i: (0, 0)),
            pl.BlockSpec((btot, 1), lambda i: (0, 0)),
        ],
        out_specs=pl.BlockSpec((na_rows, tb), lambda i: (0, i)),
        compiler_params=pltpu.CompilerParams(
            dimension_semantics=("parallel",)),
    )(x32, w_gate.astype(jnp.float32), w1.astype(jnp.float32).T,
      w2.astype(jnp.float32), b_pack.astype(jnp.float32))

    return out[:, :B].T


# speedup vs baseline: 1.0200x; 1.0200x over previous
import jax
import jax.numpy as jnp
from jax.experimental import pallas as pl
from jax.experimental.pallas import tpu as pltpu

_H = 64


def _round_up(n, m):
    return ((n + m - 1) // m) * m


def _sigmoid_t(x):
    return 0.5 * jnp.tanh(x * 0.5) + 0.5


def _fused_kernel(x_ref, wg_ref, w1t_ref, w2_ref, b_ref, out_ref, *,
                  gate_rows, nh_rows, na_rows, tb, n_split):
    wg = wg_ref[...].astype(jnp.bfloat16)
    w2 = w2_ref[...].astype(jnp.bfloat16)

    b1row = jnp.transpose(b_ref[gate_rows:gate_rows + nh_rows, :], (1, 0))
    brows = jnp.where(
        jax.lax.broadcasted_iota(jnp.int32, (8, nh_rows), 0) == 0,
        b1row, 0.0).astype(jnp.bfloat16)
    w1taug = jnp.concatenate(
        [w1t_ref[...].astype(jnp.bfloat16), brows], axis=0)

    bg = b_ref[0:gate_rows, :]
    b2 = b_ref[gate_rows + nh_rows:gate_rows + nh_rows + na_rows, :]
    ts = tb // n_split
    aug = (jax.lax.broadcasted_iota(jnp.int32, (8, ts), 0) == 0)
    aug = aug.astype(jnp.bfloat16)

    for s in range(n_split):
        xb = x_ref[s * ts:(s + 1) * ts, :].astype(jnp.bfloat16)

        gates = jax.lax.dot_general(
            wg, xb, (((1,), (1,)), ((), ())),
            preferred_element_type=jnp.float32)
        gates = gates + bg

        i = _sigmoid_t(gates[0:_H, :])
        o = _sigmoid_t(gates[_H:2 * _H, :])
        g = jnp.tanh(gates[2 * _H:3 * _H, :])
        h = (o * jnp.tanh(i * g)).astype(jnp.bfloat16)

        haug = jnp.concatenate([h, aug], axis=0)

        h1 = jax.lax.dot_general(
            w1taug, haug, (((0,), (0,)), ((), ())),
            preferred_element_type=jnp.float32)
        h1 = jnp.maximum(h1.astype(jnp.bfloat16), jnp.bfloat16(0))

        out = jnp.dot(w2, h1, preferred_element_type=jnp.float32) + b2
        out_ref[:, s * ts:(s + 1) * ts] = out


def kernel(x, w_gate, w1, w2, b_pack):
    B, n_states = x.shape
    gate_rows, k_pad = w_gate.shape
    nh_rows = w1.shape[0]
    na_rows = w2.shape[0]
    btot = gate_rows + nh_rows + na_rows

    x32 = x.astype(jnp.float32)
    if k_pad != n_states:
        x32 = jnp.pad(x32, ((0, 0), (0, k_pad - n_states)))

    if B <= 1024:
        tb = _round_up(B, 8)
        n_split = 1
    else:
        tb = min(8192, _round_up(pl.cdiv(B, 2), 256))
        n_split = 4 if tb % 32 == 0 else 1
    grid_b = pl.cdiv(B, tb)

    def body(x_ref, wg_ref, w1t_ref, w2_ref, b_ref, out_ref):
        _fused_kernel(x_ref, wg_ref, w1t_ref, w2_ref, b_ref, out_ref,
                      gate_rows=gate_rows, nh_rows=nh_rows, na_rows=na_rows,
                      tb=tb, n_split=n_split)

    out = pl.pallas_call(
        body,
        out_shape=jax.ShapeDtypeStruct((na_rows, grid_b * tb), jnp.float32),
        grid=(grid_b,),
        in_specs=[
            pl.BlockSpec((tb, k_pad), lambda i: (i, 0)),
            pl.BlockSpec((gate_rows, k_pad), lambda i: (0, 0)),
            pl.BlockSpec((_H, nh_rows), lambda i: (0, 0)),
            pl.BlockSpec((na_rows, nh_rows), lambda i: (0, 0)),
            pl.BlockSpec((btot, 1), lambda i: (0, 0)),
        ],
        out_specs=pl.BlockSpec((na_rows, tb), lambda i: (0, i)),
        compiler_params=pltpu.CompilerParams(
            dimension_semantics=("parallel",)),
    )(x32, w_gate.astype(jnp.float32), w1.astype(jnp.float32).T,
      w2.astype(jnp.float32), b_pack.astype(jnp.float32))

    return out[:, :B].T


# speedup vs baseline: 1.0506x; 1.0300x over previous
import jax
import jax.numpy as jnp
from jax.experimental import pallas as pl
from jax.experimental.pallas import tpu as pltpu

_H = 64


def _round_up(n, m):
    return ((n + m - 1) // m) * m


def _sigmoid_t(x):
    return 0.5 * jnp.tanh(x * 0.5) + 0.5


def _fused_kernel(x_ref, wg_ref, w1t_ref, w2_ref, b_ref, out_ref, *,
                  gate_rows, nh_rows, na_rows, tb, n_split):
    wg = wg_ref[...].astype(jnp.bfloat16)
    w2 = w2_ref[...].astype(jnp.bfloat16)

    b1row = b_ref[:, gate_rows:gate_rows + nh_rows]
    brows = jnp.where(
        jax.lax.broadcasted_iota(jnp.int32, (8, nh_rows), 0) == 0,
        b1row, 0.0).astype(jnp.bfloat16)
    w1taug = jnp.concatenate(
        [w1t_ref[...].astype(jnp.bfloat16), brows], axis=0)

    bg = jnp.transpose(b_ref[:, 0:gate_rows], (1, 0))
    b2 = jnp.transpose(
        b_ref[:, gate_rows + nh_rows:gate_rows + nh_rows + na_rows], (1, 0))
    ts = tb // n_split
    aug = (jax.lax.broadcasted_iota(jnp.int32, (8, ts), 0) == 0)
    aug = aug.astype(jnp.bfloat16)

    for s in range(n_split):
        xb = x_ref[s * ts:(s + 1) * ts, :].astype(jnp.bfloat16)

        gates = jax.lax.dot_general(
            wg, xb, (((1,), (1,)), ((), ())),
            preferred_element_type=jnp.float32)
        gates = gates + bg

        i = _sigmoid_t(gates[0:_H, :])
        o = _sigmoid_t(gates[_H:2 * _H, :])
        g = jnp.tanh(gates[2 * _H:3 * _H, :])
        h = (o * jnp.tanh(i * g)).astype(jnp.bfloat16)

        haug = jnp.concatenate([h, aug], axis=0)

        h1 = jax.lax.dot_general(
            w1taug, haug, (((0,), (0,)), ((), ())),
            preferred_element_type=jnp.float32)
        h1 = jnp.maximum(h1.astype(jnp.bfloat16), jnp.bfloat16(0))

        out = jnp.dot(w2, h1, preferred_element_type=jnp.float32) + b2
        out_ref[:, s * ts:(s + 1) * ts] = out


def kernel(x, w_gate, w1, w2, b_pack):
    B, n_states = x.shape
    gate_rows, k_pad = w_gate.shape
    nh_rows = w1.shape[0]
    na_rows = w2.shape[0]
    btot = gate_rows + nh_rows + na_rows

    x32 = x.astype(jnp.float32)
    if k_pad != n_states:
        x32 = jnp.pad(x32, ((0, 0), (0, k_pad - n_states)))

    if B <= 1024:
        tb = _round_up(B, 8)
        n_split = 1
    else:
        tb = min(8192, _round_up(pl.cdiv(B, 2), 256))
        n_split = 4 if tb % 32 == 0 else 1
    grid_b = pl.cdiv(B, tb)

    def body(x_ref, wg_ref, w1t_ref, w2_ref, b_ref, out_ref):
        _fused_kernel(x_ref, wg_ref, w1t_ref, w2_ref, b_ref, out_ref,
                      gate_rows=gate_rows, nh_rows=nh_rows, na_rows=na_rows,
                      tb=tb, n_split=n_split)

    out = pl.pallas_call(
        body,
        out_shape=jax.ShapeDtypeStruct((na_rows, grid_b * tb), jnp.float32),
        grid=(grid_b,),
        in_specs=[
            pl.BlockSpec((tb, k_pad), lambda i: (i, 0)),
            pl.BlockSpec((gate_rows, k_pad), lambda i: (0, 0)),
            pl.BlockSpec((_H, nh_rows), lambda i: (0, 0)),
            pl.BlockSpec((na_rows, nh_rows), lambda i: (0, 0)),
            pl.BlockSpec((1, btot), lambda i: (0, 0)),
        ],
        out_specs=pl.BlockSpec((na_rows, tb), lambda i: (0, i)),
        compiler_params=pltpu.CompilerParams(
            dimension_semantics=("parallel",)),
    )(x32, w_gate.astype(jnp.float32), w1.astype(jnp.float32).T,
      w2.astype(jnp.float32), b_pack.astype(jnp.float32).reshape(1, btot))

    return out[:, :B].T


# speedup vs baseline: 1.0553x; 1.0045x over previous
import jax
import jax.numpy as jnp
from jax.experimental import pallas as pl
from jax.experimental.pallas import tpu as pltpu

_H = 64


def _round_up(n, m):
    return ((n + m - 1) // m) * m


def _sigmoid_t(x):
    return 0.5 * jnp.tanh(x * 0.5) + 0.5


def _fused_kernel(x_ref, wg_ref, w1t_ref, w2_ref, b_ref, out_ref, *,
                  gate_rows, nh_rows, na_rows, tb, n_split):
    wg = wg_ref[...].astype(jnp.bfloat16)
    w2 = w2_ref[...].astype(jnp.bfloat16)

    b1row = b_ref[:, gate_rows:gate_rows + nh_rows]
    brows = jnp.where(
        jax.lax.broadcasted_iota(jnp.int32, (8, nh_rows), 0) == 0,
        b1row, 0.0).astype(jnp.bfloat16)
    w1taug = jnp.concatenate(
        [w1t_ref[...].astype(jnp.bfloat16), brows], axis=0)

    bg = jnp.transpose(b_ref[:, 0:gate_rows], (1, 0))
    b2 = jnp.transpose(
        b_ref[:, gate_rows + nh_rows:gate_rows + nh_rows + na_rows], (1, 0))
    ts = tb // n_split
    aug = (jax.lax.broadcasted_iota(jnp.int32, (8, ts), 0) == 0)
    aug = aug.astype(jnp.bfloat16)

    for s in range(n_split):
        xb = x_ref[s * ts:(s + 1) * ts, :].astype(jnp.bfloat16)

        gates = jax.lax.dot_general(
            wg, xb, (((1,), (1,)), ((), ())),
            preferred_element_type=jnp.float32)
        gates = gates + bg

        i = _sigmoid_t(gates[0:_H, :])
        o = _sigmoid_t(gates[_H:2 * _H, :])
        g = jnp.tanh(gates[2 * _H:3 * _H, :])
        h = (o * jnp.tanh(i * g)).astype(jnp.bfloat16)

        haug = jnp.concatenate([h, aug], axis=0)

        h1 = jax.lax.dot_general(
            w1taug, haug, (((0,), (0,)), ((), ())),
            preferred_element_type=jnp.float32)
        h1 = jnp.maximum(h1.astype(jnp.bfloat16), jnp.bfloat16(0))

        out = jnp.dot(w2, h1, preferred_element_type=jnp.float32) + b2
        out_ref[:, s * ts:(s + 1) * ts] = out


def kernel(x, w_gate, w1, w2, b_pack):
    B, n_states = x.shape
    gate_rows, k_pad = w_gate.shape
    nh_rows = w1.shape[0]
    na_rows = w2.shape[0]
    btot = gate_rows + nh_rows + na_rows

    x32 = x.astype(jnp.float32)
    if k_pad != n_states:
        x32 = jnp.pad(x32, ((0, 0), (0, k_pad - n_states)))

    if B <= 1024:
        tb = _round_up(B, 8)
        n_split = 1
    else:
        tb = min(8192, _round_up(pl.cdiv(B, 2), 256))
        n_split = 4 if tb % 32 == 0 else 1
    grid_b = pl.cdiv(B, tb)

    def body(x_ref, wg_ref, w1t_ref, w2_ref, b_ref, out_ref):
        _fused_kernel(x_ref, wg_ref, w1t_ref, w2_ref, b_ref, out_ref,
                      gate_rows=gate_rows, nh_rows=nh_rows, na_rows=na_rows,
                      tb=tb, n_split=n_split)

    out = pl.pallas_call(
        body,
        out_shape=jax.ShapeDtypeStruct((na_rows, grid_b * tb), jnp.float32),
        grid=(grid_b,),
        in_specs=[
            pl.BlockSpec((tb, k_pad), lambda i: (i, 0)),
            pl.BlockSpec((gate_rows, k_pad), lambda i: (0, 0)),
            pl.BlockSpec((_H, nh_rows), lambda i: (0, 0)),
            pl.BlockSpec((na_rows, nh_rows), lambda i: (0, 0)),
            pl.BlockSpec((1, btot), lambda i: (0, 0)),
        ],
        out_specs=pl.BlockSpec((na_rows, tb), lambda i: (0, i)),
        compiler_params=pltpu.CompilerParams(
            dimension_semantics=("arbitrary",)),
    )(x32, w_gate.astype(jnp.float32), w1.astype(jnp.float32).T,
      w2.astype(jnp.float32), b_pack.astype(jnp.float32).reshape(1, btot))

    return out[:, :B].T
